# baseline (speedup 1.0000x reference)
import functools

import jax
import jax.numpy as jnp
from jax import lax
from jax.experimental import pallas as pl
from jax.experimental.pallas import tpu as pltpu
from jax.experimental.pallas import tpu_sc as plsc

VOCAB = 30000
EMB = 125
HID = 100
OUT = 2
B = 1024
L = 200

DPAD = 128
HPAD = 128
G3 = 3 * HPAD

NC = 2
NS = 16
NW = NC * NS
ROWS = L * B
RPW = ROWS // NW
CH = 128
NCH = RPW // CH


DW = DPAD // 2


def _sc_gather(table, idx):
    mesh = plsc.VectorSubcoreMesh(core_axis_name="c", subcore_axis_name="s")

    @functools.partial(
        pl.kernel,
        mesh=mesh,
        compiler_params=pltpu.CompilerParams(use_tc_tiling_on_sc=False),
        out_type=jax.ShapeDtypeStruct((ROWS, DW), jnp.int32),
        scratch_types=[
            pltpu.VMEM((NCH, CH), jnp.int32),
            pltpu.VMEM((CH, DW), jnp.int32),
            pltpu.SemaphoreType.DMA,
        ],
    )
    def gather_kernel(table_hbm, idx_hbm, out_hbm, idx_v, buf, sem):
        wid = lax.axis_index("s") * NC + lax.axis_index("c")
        base = wid * RPW
        pltpu.sync_copy(idx_hbm.at[wid], idx_v)

        def body(c, carry):
            pltpu.async_copy(table_hbm.at[idx_v.at[c]], buf, sem).wait()
            pltpu.sync_copy(buf, out_hbm.at[pl.ds(base + c * CH, CH)])
            return carry

        lax.fori_loop(0, NCH, body, 0)

    return gather_kernel(table, idx)


def _gru_scan_body(e_ref, wih_ref, whh_ref, bih_ref, bhh_ref, wd_ref, bd_ref,
                   out_ref, h_ref):
    t = pl.program_id(0)

    @pl.when(t == 0)
    def _():
        h_ref[...] = jnp.zeros_like(h_ref)

    h = h_ref[...]
    ew = e_ref[0]
    lo = lax.bitcast_convert_type(ew << 16, jnp.float32)
    hi = lax.bitcast_convert_type(ew & jnp.int32(-65536), jnp.float32)
    e_t = jnp.concatenate([lo, hi], axis=1).astype(jnp.bfloat16)
    gi = jnp.dot(e_t, wih_ref[...], preferred_element_type=jnp.float32)
    gi = gi + bih_ref[...]
    gh = jnp.dot(h.astype(jnp.bfloat16), whh_ref[...],
                 preferred_element_type=jnp.float32)
    gh = gh + bhh_ref[...]
    r = jax.nn.sigmoid(gi[:, :HPAD] + gh[:, :HPAD])
    z = jax.nn.sigmoid(gi[:, HPAD:2 * HPAD] + gh[:, HPAD:2 * HPAD])
    n = jnp.tanh(gi[:, 2 * HPAD:] + r * gh[:, 2 * HPAD:])
    h_new = (1.0 - z) * n + z * h
    h_ref[...] = h_new

    @pl.when(t == L - 1)
    def _():
        a = jnp.tanh(h_new).astype(jnp.bfloat16)
        logits = jnp.dot(a, wd_ref[...], preferred_element_type=jnp.float32)
        logits = logits + bd_ref[...]
        m = jnp.max(logits, axis=-1, keepdims=True)
        p = jnp.exp(logits - m)
        p = p / jnp.sum(p, axis=-1, keepdims=True)
        out_ref[...] = p[:, :OUT]


def _gru_scan(e, wih, whh, bih, bhh, wd, bd):
    return pl.pallas_call(
        _gru_scan_body,
        grid=(L,),
        in_specs=[
            pl.BlockSpec((1, B, DW), lambda t: (t, 0, 0)),
            pl.BlockSpec((DPAD, G3), lambda t: (0, 0)),
            pl.BlockSpec((HPAD, G3), lambda t: (0, 0)),
            pl.BlockSpec((1, G3), lambda t: (0, 0)),
            pl.BlockSpec((1, G3), lambda t: (0, 0)),
            pl.BlockSpec((HPAD, HPAD), lambda t: (0, 0)),
            pl.BlockSpec((1, HPAD), lambda t: (0, 0)),
        ],
        out_specs=pl.BlockSpec((B, OUT), lambda t: (0, 0)),
        out_shape=jax.ShapeDtypeStruct((B, OUT), jnp.float32),
        scratch_shapes=[pltpu.VMEM((B, HPAD), jnp.float32)],
    )(e, wih, whh, bih, bhh, wd, bd)


def _pad_gates_2d(w, rows_to):
    rows = w.shape[0]
    parts = []
    for g in range(3):
        wg = w[:, g * HID:(g + 1) * HID]
        parts.append(jnp.pad(wg, ((0, rows_to - rows), (0, HPAD - HID))))
    return jnp.concatenate(parts, axis=1)


def _pad_gates_1d(b):
    parts = [jnp.pad(b[g * HID:(g + 1) * HID], (0, HPAD - HID))
             for g in range(3)]
    return jnp.concatenate(parts)[None, :]


def kernel(x, emb_table, W_ih, W_hh, b_ih, b_hh, W_dense, b_dense):
    idx = x.astype(jnp.int32).T.reshape(NW, NCH, CH)
    table_bf16 = jnp.pad(emb_table.astype(jnp.bfloat16),
                         ((0, 0), (0, DPAD - EMB)))
    table = lax.bitcast_convert_type(table_bf16.reshape(VOCAB, DW, 2),
                                     jnp.int32)
    e = _sc_gather(table, idx).reshape(L, B, DW)

    perm = jnp.concatenate([jnp.arange(0, DPAD, 2), jnp.arange(1, DPAD, 2)])
    wih = _pad_gates_2d(W_ih, DPAD)[perm, :].astype(jnp.bfloat16)
    whh = _pad_gates_2d(W_hh, HPAD).astype(jnp.bfloat16)
    bih = _pad_gates_1d(b_ih)
    bhh = _pad_gates_1d(b_hh)
    wd = jnp.pad(W_dense.T, ((0, HPAD - HID), (0, HPAD - OUT))).astype(jnp.bfloat16)
    bd = jnp.pad(b_dense, (0, HPAD - OUT), constant_values=-1e30)[None, :]

    return _gru_scan(e, wih, whh, bih, bhh, wd, bd)


# speedup vs baseline: 1.8536x; 1.8536x over previous
import functools

import jax
import jax.numpy as jnp
from jax import lax
from jax.experimental import pallas as pl
from jax.experimental.pallas import tpu as pltpu
from jax.experimental.pallas import tpu_sc as plsc

VOCAB = 30000
EMB = 125
HID = 100
OUT = 2
B = 1024
L = 200

DPAD = 128
HPAD = 128
G3 = 3 * HPAD
TB = 10
NT = L // TB

NC = 2
NS = 16
NW = NC * NS
ROWS = L * B
RPW = ROWS // NW
CH = 128
NCH = RPW // CH


def _sc_gather(table, idx):
    mesh = plsc.VectorSubcoreMesh(core_axis_name="c", subcore_axis_name="s")

    @functools.partial(
        pl.kernel,
        mesh=mesh,
        out_type=jax.ShapeDtypeStruct((ROWS, DPAD), jnp.float32),
        scratch_types=[
            pltpu.VMEM((NCH, CH), jnp.int32),
            pltpu.VMEM((CH, DPAD), jnp.float32),
            pltpu.SemaphoreType.DMA,
        ],
    )
    def gather_kernel(table_hbm, idx_hbm, out_hbm, idx_v, buf, sem):
        wid = lax.axis_index("s") * NC + lax.axis_index("c")
        base = wid * RPW
        pltpu.sync_copy(idx_hbm.at[wid], idx_v)

        def body(c, carry):
            pltpu.async_copy(table_hbm.at[idx_v.at[c]], buf, sem).wait()
            pltpu.sync_copy(buf, out_hbm.at[pl.ds(base + c * CH, CH)])
            return carry

        lax.fori_loop(0, NCH, body, 0)

    return gather_kernel(table, idx)


def _gru_scan_body(e_ref, wih_ref, whh_ref, bih_ref, bhh_ref, wd_ref, bd_ref,
                   out_ref, h_ref):
    g = pl.program_id(0)

    @pl.when(g == 0)
    def _():
        h_ref[...] = jnp.zeros_like(h_ref)

    e_blk = e_ref[...].reshape(TB * B, DPAD).astype(jnp.bfloat16)
    gi_all = jnp.dot(e_blk, wih_ref[...], preferred_element_type=jnp.float32)

    h = h_ref[...]
    for t in range(TB):
        gi = gi_all[t * B:(t + 1) * B] + bih_ref[...]
        gh = jnp.dot(h.astype(jnp.bfloat16), whh_ref[...],
                     preferred_element_type=jnp.float32)
        gh = gh + bhh_ref[...]
        r = jax.nn.sigmoid(gi[:, :HPAD] + gh[:, :HPAD])
        z = jax.nn.sigmoid(gi[:, HPAD:2 * HPAD] + gh[:, HPAD:2 * HPAD])
        n = jnp.tanh(gi[:, 2 * HPAD:] + r * gh[:, 2 * HPAD:])
        h = (1.0 - z) * n + z * h
    h_ref[...] = h

    @pl.when(g == NT - 1)
    def _():
        a = jnp.tanh(h).astype(jnp.bfloat16)
        logits = jnp.dot(a, wd_ref[...], preferred_element_type=jnp.float32)
        logits = logits + bd_ref[...]
        m = jnp.max(logits, axis=-1, keepdims=True)
        p = jnp.exp(logits - m)
        p = p / jnp.sum(p, axis=-1, keepdims=True)
        out_ref[...] = p[:, :OUT]


def _gru_scan(e, wih, whh, bih, bhh, wd, bd):
    return pl.pallas_call(
        _gru_scan_body,
        grid=(NT,),
        in_specs=[
            pl.BlockSpec((TB, B, DPAD), lambda g: (g, 0, 0)),
            pl.BlockSpec((DPAD, G3), lambda g: (0, 0)),
            pl.BlockSpec((HPAD, G3), lambda g: (0, 0)),
            pl.BlockSpec((1, G3), lambda g: (0, 0)),
            pl.BlockSpec((1, G3), lambda g: (0, 0)),
            pl.BlockSpec((HPAD, HPAD), lambda g: (0, 0)),
            pl.BlockSpec((1, HPAD), lambda g: (0, 0)),
        ],
        out_specs=pl.BlockSpec((B, OUT), lambda g: (0, 0)),
        out_shape=jax.ShapeDtypeStruct((B, OUT), jnp.float32),
        scratch_shapes=[pltpu.VMEM((B, HPAD), jnp.float32)],
    )(e, wih, whh, bih, bhh, wd, bd)


def _pad_gates_2d(w, rows_to):
    rows = w.shape[0]
    parts = []
    for g in range(3):
        wg = w[:, g * HID:(g + 1) * HID]
        parts.append(jnp.pad(wg, ((0, rows_to - rows), (0, HPAD - HID))))
    return jnp.concatenate(parts, axis=1)


def _pad_gates_1d(b):
    parts = [jnp.pad(b[g * HID:(g + 1) * HID], (0, HPAD - HID))
             for g in range(3)]
    return jnp.concatenate(parts)[None, :]


def kernel(x, emb_table, W_ih, W_hh, b_ih, b_hh, W_dense, b_dense):
    idx = x.astype(jnp.int32).T.reshape(NW, NCH, CH)
    table = jnp.pad(emb_table, ((0, 0), (0, DPAD - EMB)))
    e = _sc_gather(table, idx).reshape(L, B, DPAD)

    wih = _pad_gates_2d(W_ih, DPAD).astype(jnp.bfloat16)
    whh = _pad_gates_2d(W_hh, HPAD).astype(jnp.bfloat16)
    bih = _pad_gates_1d(b_ih)
    bhh = _pad_gates_1d(b_hh)
    wd = jnp.pad(W_dense.T,
                 ((0, HPAD - HID), (0, HPAD - OUT))).astype(jnp.bfloat16)
    bd = jnp.pad(b_dense, (0, HPAD - OUT), constant_values=-1e30)[None, :]

    return _gru_scan(e, wih, whh, bih, bhh, wd, bd)


# speedup vs baseline: 1.9746x; 1.0653x over previous
import functools

import jax
import jax.numpy as jnp
from jax import lax
from jax.experimental import pallas as pl
from jax.experimental.pallas import tpu as pltpu
from jax.experimental.pallas import tpu_sc as plsc

VOCAB = 30000
EMB = 125
HID = 100
OUT = 2
B = 1024
L = 200

DPAD = 128
HPAD = 128
G3 = 3 * HPAD
TB = 10
NT = L // TB

NC = 2
NS = 16
NW = NC * NS
ROWS = L * B
RPW = ROWS // NW
CH = 128
NCH = RPW // CH


def _sc_gather(table, idx):
    mesh = plsc.VectorSubcoreMesh(core_axis_name="c", subcore_axis_name="s")

    @functools.partial(
        pl.kernel,
        mesh=mesh,
        out_type=jax.ShapeDtypeStruct((ROWS, DPAD), jnp.float32),
        scratch_types=[
            pltpu.VMEM((NCH, CH), jnp.int32),
            pltpu.VMEM((CH, DPAD), jnp.float32),
            pltpu.VMEM((CH, DPAD), jnp.float32),
            pltpu.SemaphoreType.DMA,
            pltpu.SemaphoreType.DMA,
        ],
    )
    def gather_kernel(table_hbm, idx_hbm, out_hbm, idx_v, buf0, buf1, sem0,
                      sem1):
        wid = lax.axis_index("s") * NC + lax.axis_index("c")
        base = wid * RPW
        pltpu.sync_copy(idx_hbm.at[wid], idx_v)

        pltpu.async_copy(table_hbm.at[idx_v.at[0]], buf0, sem0)

        def body(i, carry):
            c0 = 2 * i
            c1 = c0 + 1
            pltpu.make_async_copy(table_hbm.at[idx_v.at[c0]], buf0,
                                  sem0).wait()
            pltpu.async_copy(table_hbm.at[idx_v.at[c1]], buf1, sem1)
            pltpu.sync_copy(buf0, out_hbm.at[pl.ds(base + c0 * CH, CH)])
            pltpu.make_async_copy(table_hbm.at[idx_v.at[c1]], buf1,
                                  sem1).wait()

            @pl.when(c1 + 1 < NCH)
            def _():
                pltpu.async_copy(table_hbm.at[idx_v.at[c1 + 1]], buf0, sem0)

            pltpu.sync_copy(buf1, out_hbm.at[pl.ds(base + c1 * CH, CH)])
            return carry

        lax.fori_loop(0, NCH // 2, body, 0)

    return gather_kernel(table, idx)


def _gru_scan_body(e_ref, wih_ref, whh_ref, bih_ref, bhh_ref, wd_ref, bd_ref,
                   out_ref, h_ref):
    g = pl.program_id(0)

    @pl.when(g == 0)
    def _():
        h_ref[...] = jnp.zeros_like(h_ref)

    e_blk = e_ref[...].reshape(TB * B, DPAD).astype(jnp.bfloat16)
    gi_all = jnp.dot(e_blk, wih_ref[...], preferred_element_type=jnp.float32)

    h = h_ref[...]
    for t in range(TB):
        gi = gi_all[t * B:(t + 1) * B] + bih_ref[...]
        gh = jnp.dot(h.astype(jnp.bfloat16), whh_ref[...],
                     preferred_element_type=jnp.float32)
        gh = gh + bhh_ref[...]
        r = jax.nn.sigmoid(gi[:, :HPAD] + gh[:, :HPAD])
        z = jax.nn.sigmoid(gi[:, HPAD:2 * HPAD] + gh[:, HPAD:2 * HPAD])
        n = jnp.tanh(gi[:, 2 * HPAD:] + r * gh[:, 2 * HPAD:])
        h = (1.0 - z) * n + z * h
    h_ref[...] = h

    @pl.when(g == NT - 1)
    def _():
        a = jnp.tanh(h).astype(jnp.bfloat16)
        logits = jnp.dot(a, wd_ref[...], preferred_element_type=jnp.float32)
        logits = logits + bd_ref[...]
        m = jnp.max(logits, axis=-1, keepdims=True)
        p = jnp.exp(logits - m)
        p = p / jnp.sum(p, axis=-1, keepdims=True)
        out_ref[...] = p[:, :OUT]


def _gru_scan(e, wih, whh, bih, bhh, wd, bd):
    return pl.pallas_call(
        _gru_scan_body,
        grid=(NT,),
        in_specs=[
            pl.BlockSpec((TB, B, DPAD), lambda g: (g, 0, 0)),
            pl.BlockSpec((DPAD, G3), lambda g: (0, 0)),
            pl.BlockSpec((HPAD, G3), lambda g: (0, 0)),
            pl.BlockSpec((1, G3), lambda g: (0, 0)),
            pl.BlockSpec((1, G3), lambda g: (0, 0)),
            pl.BlockSpec((HPAD, HPAD), lambda g: (0, 0)),
            pl.BlockSpec((1, HPAD), lambda g: (0, 0)),
        ],
        out_specs=pl.BlockSpec((B, OUT), lambda g: (0, 0)),
        out_shape=jax.ShapeDtypeStruct((B, OUT), jnp.float32),
        scratch_shapes=[pltpu.VMEM((B, HPAD), jnp.float32)],
    )(e, wih, whh, bih, bhh, wd, bd)


def _pad_gates_2d(w, rows_to):
    rows = w.shape[0]
    parts = []
    for g in range(3):
        wg = w[:, g * HID:(g + 1) * HID]
        parts.append(jnp.pad(wg, ((0, rows_to - rows), (0, HPAD - HID))))
    return jnp.concatenate(parts, axis=1)


def _pad_gates_1d(b):
    parts = [jnp.pad(b[g * HID:(g + 1) * HID], (0, HPAD - HID))
             for g in range(3)]
    return jnp.concatenate(parts)[None, :]


def kernel(x, emb_table, W_ih, W_hh, b_ih, b_hh, W_dense, b_dense):
    idx = x.astype(jnp.int32).T.reshape(NW, NCH, CH)
    table = jnp.pad(emb_table, ((0, 0), (0, DPAD - EMB)))
    e = _sc_gather(table, idx).reshape(L, B, DPAD)

    wih = _pad_gates_2d(W_ih, DPAD).astype(jnp.bfloat16)
    whh = _pad_gates_2d(W_hh, HPAD).astype(jnp.bfloat16)
    bih = _pad_gates_1d(b_ih)
    bhh = _pad_gates_1d(b_hh)
    wd = jnp.pad(W_dense.T,
                 ((0, HPAD - HID), (0, HPAD - OUT))).astype(jnp.bfloat16)
    bd = jnp.pad(b_dense, (0, HPAD - OUT), constant_values=-1e30)[None, :]

    return _gru_scan(e, wih, whh, bih, bhh, wd, bd)


# speedup vs baseline: 2.0530x; 1.0397x over previous
import functools

import jax
import jax.numpy as jnp
from jax import lax
from jax.experimental import pallas as pl
from jax.experimental.pallas import tpu as pltpu
from jax.experimental.pallas import tpu_sc as plsc

VOCAB = 30000
EMB = 125
HID = 100
OUT = 2
B = 1024
L = 200

DPAD = 128
HPAD = 128
G3 = 3 * HPAD
TB = 10
NT = L // TB

NC = 2
NS = 16
NW = NC * NS
ROWS = L * B
RPW = ROWS // NW
CH = 128
NCH = RPW // CH


def _sc_gather(table, idx):
    mesh = plsc.VectorSubcoreMesh(core_axis_name="c", subcore_axis_name="s")

    @functools.partial(
        pl.kernel,
        mesh=mesh,
        out_type=jax.ShapeDtypeStruct((ROWS, DPAD), jnp.float32),
        scratch_types=[
            pltpu.VMEM((NCH, CH), jnp.int32),
            pltpu.VMEM((CH, DPAD), jnp.float32),
            pltpu.VMEM((CH, DPAD), jnp.float32),
            pltpu.SemaphoreType.DMA,
            pltpu.SemaphoreType.DMA,
        ],
    )
    def gather_kernel(table_hbm, idx_hbm, out_hbm, idx_v, buf0, buf1, sem0,
                      sem1):
        wid = lax.axis_index("s") * NC + lax.axis_index("c")
        base = wid * RPW
        pltpu.sync_copy(idx_hbm.at[wid], idx_v)

        pltpu.async_copy(table_hbm.at[idx_v.at[0]], buf0, sem0)

        def body(i, carry):
            c0 = 2 * i
            c1 = c0 + 1
            pltpu.make_async_copy(table_hbm.at[idx_v.at[c0]], buf0,
                                  sem0).wait()
            pltpu.async_copy(table_hbm.at[idx_v.at[c1]], buf1, sem1)
            pltpu.sync_copy(buf0, out_hbm.at[pl.ds(base + c0 * CH, CH)])
            pltpu.make_async_copy(table_hbm.at[idx_v.at[c1]], buf1,
                                  sem1).wait()

            @pl.when(c1 + 1 < NCH)
            def _():
                pltpu.async_copy(table_hbm.at[idx_v.at[c1 + 1]], buf0, sem0)

            pltpu.sync_copy(buf1, out_hbm.at[pl.ds(base + c1 * CH, CH)])
            return carry

        lax.fori_loop(0, NCH // 2, body, 0)

    return gather_kernel(table, idx)


def _gru_scan_body(e_ref, wih_ref, whh_ref, bih_ref, bhh_ref, wd_ref, bd_ref,
                   out_ref, h_ref):
    g = pl.program_id(0)

    @pl.when(g == 0)
    def _():
        h_ref[...] = jnp.zeros_like(h_ref)

    e_blk = e_ref[...].reshape(TB * B, DPAD).astype(jnp.bfloat16)
    gi_all = jnp.dot(e_blk, wih_ref[...],
                     preferred_element_type=jnp.float32).astype(jnp.bfloat16)

    h = h_ref[...]
    for t in range(TB):
        gi = gi_all[t * B:(t + 1) * B] + bih_ref[...]
        gh = jnp.dot(h, whh_ref[...],
                     preferred_element_type=jnp.float32).astype(jnp.bfloat16)
        gh = gh + bhh_ref[...]
        r = jax.nn.sigmoid(gi[:, :HPAD] + gh[:, :HPAD])
        z = jax.nn.sigmoid(gi[:, HPAD:2 * HPAD] + gh[:, HPAD:2 * HPAD])
        n = jnp.tanh(gi[:, 2 * HPAD:] + r * gh[:, 2 * HPAD:])
        h = ((1.0 - z) * n + z * h).astype(jnp.bfloat16)
    h_ref[...] = h

    @pl.when(g == NT - 1)
    def _():
        a = jnp.tanh(h.astype(jnp.float32)).astype(jnp.bfloat16)
        logits = jnp.dot(a, wd_ref[...], preferred_element_type=jnp.float32)
        logits = logits + bd_ref[...]
        m = jnp.max(logits, axis=-1, keepdims=True)
        p = jnp.exp(logits - m)
        p = p / jnp.sum(p, axis=-1, keepdims=True)
        out_ref[...] = p[:, :OUT]


def _gru_scan(e, wih, whh, bih, bhh, wd, bd):
    return pl.pallas_call(
        _gru_scan_body,
        grid=(NT,),
        in_specs=[
            pl.BlockSpec((TB, B, DPAD), lambda g: (g, 0, 0)),
            pl.BlockSpec((DPAD, G3), lambda g: (0, 0)),
            pl.BlockSpec((HPAD, G3), lambda g: (0, 0)),
            pl.BlockSpec((1, G3), lambda g: (0, 0)),
            pl.BlockSpec((1, G3), lambda g: (0, 0)),
            pl.BlockSpec((HPAD, HPAD), lambda g: (0, 0)),
            pl.BlockSpec((1, HPAD), lambda g: (0, 0)),
        ],
        out_specs=pl.BlockSpec((B, OUT), lambda g: (0, 0)),
        out_shape=jax.ShapeDtypeStruct((B, OUT), jnp.float32),
        scratch_shapes=[pltpu.VMEM((B, HPAD), jnp.bfloat16)],
    )(e, wih, whh, bih, bhh, wd, bd)


def _pad_gates_2d(w, rows_to):
    rows = w.shape[0]
    parts = []
    for g in range(3):
        wg = w[:, g * HID:(g + 1) * HID]
        parts.append(jnp.pad(wg, ((0, rows_to - rows), (0, HPAD - HID))))
    return jnp.concatenate(parts, axis=1)


def _pad_gates_1d(b):
    parts = [jnp.pad(b[g * HID:(g + 1) * HID], (0, HPAD - HID))
             for g in range(3)]
    return jnp.concatenate(parts)[None, :]


def kernel(x, emb_table, W_ih, W_hh, b_ih, b_hh, W_dense, b_dense):
    idx = x.astype(jnp.int32).T.reshape(NW, NCH, CH)
    table = jnp.pad(emb_table, ((0, 0), (0, DPAD - EMB)))
    e = _sc_gather(table, idx).reshape(L, B, DPAD)

    wih = _pad_gates_2d(W_ih, DPAD).astype(jnp.bfloat16)
    whh = _pad_gates_2d(W_hh, HPAD).astype(jnp.bfloat16)
    bih = _pad_gates_1d(b_ih).astype(jnp.bfloat16)
    bhh = _pad_gates_1d(b_hh).astype(jnp.bfloat16)
    wd = jnp.pad(W_dense.T,
                 ((0, HPAD - HID), (0, HPAD - OUT))).astype(jnp.bfloat16)
    bd = jnp.pad(b_dense, (0, HPAD - OUT), constant_values=-1e30)[None, :]

    return _gru_scan(e, wih, whh, bih, bhh, wd, bd)


# speedup vs baseline: 2.3006x; 1.1206x over previous
import functools

import jax
import jax.numpy as jnp
from jax import lax
from jax.experimental import pallas as pl
from jax.experimental.pallas import tpu as pltpu
from jax.experimental.pallas import tpu_sc as plsc

VOCAB = 30000
EMB = 125
HID = 100
OUT = 2
B = 1024
L = 200

DPAD = 128
HPAD = 128
G3 = 3 * HPAD
TB = 10
NT = L // TB

NC = 2
NS = 16
NW = NC * NS
ROWS = L * B
RPW = ROWS // NW
CH = 128
NCH = RPW // CH


def _sc_gather(table, idx):
    mesh = plsc.VectorSubcoreMesh(core_axis_name="c", subcore_axis_name="s")

    @functools.partial(
        pl.kernel,
        mesh=mesh,
        out_type=jax.ShapeDtypeStruct((ROWS, DPAD), jnp.float32),
        scratch_types=[
            pltpu.VMEM((NCH, CH), jnp.int32),
            pltpu.VMEM((CH, DPAD), jnp.float32),
            pltpu.VMEM((CH, DPAD), jnp.float32),
            pltpu.SemaphoreType.DMA,
            pltpu.SemaphoreType.DMA,
        ],
    )
    def gather_kernel(table_hbm, idx_hbm, out_hbm, idx_v, buf0, buf1, sem0,
                      sem1):
        wid = lax.axis_index("s") * NC + lax.axis_index("c")
        base = wid * RPW
        pltpu.sync_copy(idx_hbm.at[wid], idx_v)

        pltpu.async_copy(table_hbm.at[idx_v.at[0]], buf0, sem0)

        def body(i, carry):
            c0 = 2 * i
            c1 = c0 + 1
            pltpu.make_async_copy(table_hbm.at[idx_v.at[c0]], buf0,
                                  sem0).wait()
            pltpu.async_copy(table_hbm.at[idx_v.at[c1]], buf1, sem1)
            pltpu.sync_copy(buf0, out_hbm.at[pl.ds(base + c0 * CH, CH)])
            pltpu.make_async_copy(table_hbm.at[idx_v.at[c1]], buf1,
                                  sem1).wait()

            @pl.when(c1 + 1 < NCH)
            def _():
                pltpu.async_copy(table_hbm.at[idx_v.at[c1 + 1]], buf0, sem0)

            pltpu.sync_copy(buf1, out_hbm.at[pl.ds(base + c1 * CH, CH)])
            return carry

        lax.fori_loop(0, NCH // 2, body, 0)

    return gather_kernel(table, idx)


def _gru_scan_body(e_ref, wih_ref, whh_ref, bhn_ref, wd_ref, bd_ref,
                   out_ref, h_ref):
    g = pl.program_id(0)

    @pl.when(g == 0)
    def _():
        h_ref[...] = jnp.zeros_like(h_ref)

    e_blk = e_ref[...].reshape(TB * B, DPAD).astype(jnp.bfloat16)
    gi_all = jnp.dot(e_blk, wih_ref[...],
                     preferred_element_type=jnp.float32).astype(jnp.bfloat16)

    half = jnp.bfloat16(0.5)
    h = h_ref[...]
    for t in range(TB):
        gi = gi_all[t * B:(t + 1) * B]
        gh = jnp.dot(h, whh_ref[...],
                     preferred_element_type=jnp.float32).astype(jnp.bfloat16)
        rt = jnp.tanh(half * (gi[:, :HPAD] + gh[:, :HPAD]))
        zt = jnp.tanh(half * (gi[:, HPAD:2 * HPAD] + gh[:, HPAD:2 * HPAD]))
        hn = gh[:, 2 * HPAD:] + bhn_ref[...]
        n = jnp.tanh(gi[:, 2 * HPAD:] + (half * rt + half) * hn)
        h = (half * ((n + h) + zt * (h - n))).astype(jnp.bfloat16)
    h_ref[...] = h

    @pl.when(g == NT - 1)
    def _():
        a = jnp.tanh(h.astype(jnp.float32)).astype(jnp.bfloat16)
        logits = jnp.dot(a, wd_ref[...], preferred_element_type=jnp.float32)
        logits = logits + bd_ref[...]
        m = jnp.max(logits, axis=-1, keepdims=True)
        p = jnp.exp(logits - m)
        p = p / jnp.sum(p, axis=-1, keepdims=True)
        out_ref[...] = p[:, :OUT]


def _gru_scan(e, wih, whh, bhn, wd, bd):
    return pl.pallas_call(
        _gru_scan_body,
        grid=(NT,),
        in_specs=[
            pl.BlockSpec((TB, B, DPAD), lambda g: (g, 0, 0)),
            pl.BlockSpec((DPAD, G3), lambda g: (0, 0)),
            pl.BlockSpec((HPAD, G3), lambda g: (0, 0)),
            pl.BlockSpec((1, HPAD), lambda g: (0, 0)),
            pl.BlockSpec((HPAD, HPAD), lambda g: (0, 0)),
            pl.BlockSpec((1, HPAD), lambda g: (0, 0)),
        ],
        out_specs=pl.BlockSpec((B, OUT), lambda g: (0, 0)),
        out_shape=jax.ShapeDtypeStruct((B, OUT), jnp.float32),
        scratch_shapes=[pltpu.VMEM((B, HPAD), jnp.bfloat16)],
    )(e, wih, whh, bhn, wd, bd)


def _pad_gates_2d(w, rows_to):
    rows = w.shape[0]
    parts = []
    for g in range(3):
        wg = w[:, g * HID:(g + 1) * HID]
        parts.append(jnp.pad(wg, ((0, rows_to - rows), (0, HPAD - HID))))
    return jnp.concatenate(parts, axis=1)


def _pad_gates_1d(b):
    parts = [jnp.pad(b[g * HID:(g + 1) * HID], (0, HPAD - HID))
             for g in range(3)]
    return jnp.concatenate(parts)[None, :]


def kernel(x, emb_table, W_ih, W_hh, b_ih, b_hh, W_dense, b_dense):
    idx = x.astype(jnp.int32).T.reshape(NW, NCH, CH)
    table = jnp.concatenate(
        [emb_table,
         jnp.ones((VOCAB, 1), jnp.float32),
         jnp.zeros((VOCAB, DPAD - EMB - 1), jnp.float32)], axis=1)
    e = _sc_gather(table, idx).reshape(L, B, DPAD)

    b_comb = b_ih + jnp.concatenate(
        [b_hh[:2 * HID], jnp.zeros((HID,), jnp.float32)])
    wih = jnp.concatenate(
        [_pad_gates_2d(W_ih, EMB),
         _pad_gates_1d(b_comb),
         jnp.zeros((DPAD - EMB - 1, G3), jnp.float32)],
        axis=0).astype(jnp.bfloat16)
    whh = _pad_gates_2d(W_hh, HPAD).astype(jnp.bfloat16)
    bhn = jnp.pad(b_hh[2 * HID:],
                  (0, HPAD - HID))[None, :].astype(jnp.bfloat16)
    wd = jnp.pad(W_dense.T,
                 ((0, HPAD - HID), (0, HPAD - OUT))).astype(jnp.bfloat16)
    bd = jnp.pad(b_dense, (0, HPAD - OUT), constant_values=-1e30)[None, :]

    return _gru_scan(e, wih, whh, bhn, wd, bd)


# speedup vs baseline: 2.5607x; 1.1131x over previous
import functools

import jax
import jax.numpy as jnp
from jax import lax
from jax.experimental import pallas as pl
from jax.experimental.pallas import tpu as pltpu
from jax.experimental.pallas import tpu_sc as plsc

VOCAB = 30000
EMB = 125
HID = 100
OUT = 2
B = 1024
L = 200

DPAD = 128
HPAD = 128
G3 = 3 * HPAD
TB = 10
NT = L // TB

NC = 2
NS = 16
NW = NC * NS
CH = 128


def _sc_gather(table, idx, rows):
    rpw = rows // NW
    nch = rpw // CH
    mesh = plsc.VectorSubcoreMesh(core_axis_name="c", subcore_axis_name="s")

    @functools.partial(
        pl.kernel,
        mesh=mesh,
        out_type=jax.ShapeDtypeStruct((rows, DPAD), jnp.float32),
        scratch_types=[
            pltpu.VMEM((nch, CH), jnp.int32),
            pltpu.VMEM((CH, DPAD), jnp.float32),
            pltpu.VMEM((CH, DPAD), jnp.float32),
            pltpu.SemaphoreType.DMA,
            pltpu.SemaphoreType.DMA,
        ],
    )
    def gather_kernel(table_hbm, idx_hbm, out_hbm, idx_v, buf0, buf1, sem0,
                      sem1):
        wid = lax.axis_index("s") * NC + lax.axis_index("c")
        base = wid * rpw
        pltpu.sync_copy(idx_hbm.at[wid], idx_v)

        pltpu.async_copy(table_hbm.at[idx_v.at[0]], buf0, sem0)

        def body(i, carry):
            c0 = 2 * i
            c1 = c0 + 1
            pltpu.make_async_copy(table_hbm.at[idx_v.at[c0]], buf0,
                                  sem0).wait()
            pltpu.async_copy(table_hbm.at[idx_v.at[c1]], buf1, sem1)
            pltpu.sync_copy(buf0, out_hbm.at[pl.ds(base + c0 * CH, CH)])
            pltpu.make_async_copy(table_hbm.at[idx_v.at[c1]], buf1,
                                  sem1).wait()

            @pl.when(c1 + 1 < nch)
            def _():
                pltpu.async_copy(table_hbm.at[idx_v.at[c1 + 1]], buf0, sem0)

            pltpu.sync_copy(buf1, out_hbm.at[pl.ds(base + c1 * CH, CH)])
            return carry

        lax.fori_loop(0, nch // 2, body, 0)

        if nch % 2 == 1:
            c_last = nch - 1
            pltpu.make_async_copy(table_hbm.at[idx_v.at[c_last]], buf0,
                                  sem0).wait()
            pltpu.sync_copy(buf0, out_hbm.at[pl.ds(base + c_last * CH, CH)])

    return gather_kernel(table, idx)


def _make_scan_body(bh):
    def _gru_scan_body(e_ref, wih_ref, whh_ref, bhn_ref, wd_ref, bd_ref,
                       out_ref, h_ref):
        g = pl.program_id(0)

        @pl.when(g == 0)
        def _():
            h_ref[...] = jnp.zeros_like(h_ref)

        e_blk = e_ref[...].reshape(TB * bh, DPAD).astype(jnp.bfloat16)
        gi_all = jnp.dot(e_blk, wih_ref[...],
                         preferred_element_type=jnp.float32
                         ).astype(jnp.bfloat16)

        half = jnp.bfloat16(0.5)
        h = h_ref[...]
        for t in range(TB):
            gi = gi_all[t * bh:(t + 1) * bh]
            gh = jnp.dot(h, whh_ref[...],
                         preferred_element_type=jnp.float32
                         ).astype(jnp.bfloat16)
            rt = jnp.tanh(half * (gi[:, :HPAD] + gh[:, :HPAD]))
            zt = jnp.tanh(half * (gi[:, HPAD:2 * HPAD]
                                  + gh[:, HPAD:2 * HPAD]))
            hn = gh[:, 2 * HPAD:] + bhn_ref[...]
            n = jnp.tanh(gi[:, 2 * HPAD:] + (half * rt + half) * hn)
            h = (half * ((n + h) + zt * (h - n))).astype(jnp.bfloat16)
        h_ref[...] = h

        @pl.when(g == NT - 1)
        def _():
            a = jnp.tanh(h.astype(jnp.float32)).astype(jnp.bfloat16)
            logits = jnp.dot(a, wd_ref[...],
                             preferred_element_type=jnp.float32)
            logits = logits + bd_ref[...]
            m = jnp.max(logits, axis=-1, keepdims=True)
            p = jnp.exp(logits - m)
            p = p / jnp.sum(p, axis=-1, keepdims=True)
            out_ref[...] = p[:, :OUT]

    return _gru_scan_body


def _gru_scan(e, wih, whh, bhn, wd, bd, bh):
    return pl.pallas_call(
        _make_scan_body(bh),
        grid=(NT,),
        in_specs=[
            pl.BlockSpec((TB, bh, DPAD), lambda g: (g, 0, 0)),
            pl.BlockSpec((DPAD, G3), lambda g: (0, 0)),
            pl.BlockSpec((HPAD, G3), lambda g: (0, 0)),
            pl.BlockSpec((1, HPAD), lambda g: (0, 0)),
            pl.BlockSpec((HPAD, HPAD), lambda g: (0, 0)),
            pl.BlockSpec((1, HPAD), lambda g: (0, 0)),
        ],
        out_specs=pl.BlockSpec((bh, OUT), lambda g: (0, 0)),
        out_shape=jax.ShapeDtypeStruct((bh, OUT), jnp.float32),
        scratch_shapes=[pltpu.VMEM((bh, HPAD), jnp.bfloat16)],
    )(e, wih, whh, bhn, wd, bd)


def _pad_gates_2d(w, rows_to):
    rows = w.shape[0]
    parts = []
    for g in range(3):
        wg = w[:, g * HID:(g + 1) * HID]
        parts.append(jnp.pad(wg, ((0, rows_to - rows), (0, HPAD - HID))))
    return jnp.concatenate(parts, axis=1)


def _pad_gates_1d(b):
    parts = [jnp.pad(b[g * HID:(g + 1) * HID], (0, HPAD - HID))
             for g in range(3)]
    return jnp.concatenate(parts)[None, :]


def kernel(x, emb_table, W_ih, W_hh, b_ih, b_hh, W_dense, b_dense):
    bh = B // 2
    rows_h = L * bh
    nch = rows_h // NW // CH
    xi = x.astype(jnp.int32)
    idx0 = xi[:bh].T.reshape(NW, nch, CH)
    idx1 = xi[bh:].T.reshape(NW, nch, CH)

    table = jnp.concatenate(
        [emb_table,
         jnp.ones((VOCAB, 1), jnp.float32),
         jnp.zeros((VOCAB, DPAD - EMB - 1), jnp.float32)], axis=1)

    b_comb = b_ih + jnp.concatenate(
        [b_hh[:2 * HID], jnp.zeros((HID,), jnp.float32)])
    wih = jnp.concatenate(
        [_pad_gates_2d(W_ih, EMB),
         _pad_gates_1d(b_comb),
         jnp.zeros((DPAD - EMB - 1, G3), jnp.float32)],
        axis=0).astype(jnp.bfloat16)
    whh = _pad_gates_2d(W_hh, HPAD).astype(jnp.bfloat16)
    bhn = jnp.pad(b_hh[2 * HID:],
                  (0, HPAD - HID))[None, :].astype(jnp.bfloat16)
    wd = jnp.pad(W_dense.T,
                 ((0, HPAD - HID), (0, HPAD - OUT))).astype(jnp.bfloat16)
    bd = jnp.pad(b_dense, (0, HPAD - OUT), constant_values=-1e30)[None, :]

    e0 = _sc_gather(table, idx0, rows_h).reshape(L, bh, DPAD)
    e1 = _sc_gather(table, idx1, rows_h).reshape(L, bh, DPAD)
    out0 = _gru_scan(e0, wih, whh, bhn, wd, bd, bh)
    out1 = _gru_scan(e1, wih, whh, bhn, wd, bd, bh)
    return jnp.concatenate([out0, out1], axis=0)


# speedup vs baseline: 2.5780x; 1.0068x over previous
import functools

import jax
import jax.numpy as jnp
from jax import lax
from jax.experimental import pallas as pl
from jax.experimental.pallas import tpu as pltpu
from jax.experimental.pallas import tpu_sc as plsc

VOCAB = 30000
EMB = 125
HID = 100
OUT = 2
B = 1024
L = 200

DPAD = 128
HPAD = 128
G3 = 3 * HPAD
TB = 10
NSEG = 4
LSEG = L // NSEG
NT = LSEG // TB

NC = 2
NS = 16
NW = NC * NS
SEG_ROWS = LSEG * B
RPW = SEG_ROWS // NW
CH = 80
NCH = RPW // CH


def _sc_gather(table, idx):
    mesh = plsc.VectorSubcoreMesh(core_axis_name="c", subcore_axis_name="s")

    @functools.partial(
        pl.kernel,
        mesh=mesh,
        out_type=jax.ShapeDtypeStruct((SEG_ROWS, DPAD), jnp.float32),
        scratch_types=[
            pltpu.VMEM((NCH, CH), jnp.int32),
            pltpu.VMEM((CH, DPAD), jnp.float32),
            pltpu.VMEM((CH, DPAD), jnp.float32),
            pltpu.SemaphoreType.DMA,
            pltpu.SemaphoreType.DMA,
        ],
    )
    def gather_kernel(table_hbm, idx_hbm, out_hbm, idx_v, buf0, buf1, sem0,
                      sem1):
        wid = lax.axis_index("s") * NC + lax.axis_index("c")
        base = wid * RPW
        pltpu.sync_copy(idx_hbm.at[wid], idx_v)

        pltpu.async_copy(table_hbm.at[idx_v.at[0]], buf0, sem0)

        def body(i, carry):
            c0 = 2 * i
            c1 = c0 + 1
            pltpu.make_async_copy(table_hbm.at[idx_v.at[c0]], buf0,
                                  sem0).wait()
            pltpu.async_copy(table_hbm.at[idx_v.at[c1]], buf1, sem1)
            pltpu.sync_copy(buf0, out_hbm.at[pl.ds(base + c0 * CH, CH)])
            pltpu.make_async_copy(table_hbm.at[idx_v.at[c1]], buf1,
                                  sem1).wait()

            @pl.when(c1 + 1 < NCH)
            def _():
                pltpu.async_copy(table_hbm.at[idx_v.at[c1 + 1]], buf0, sem0)

            pltpu.sync_copy(buf1, out_hbm.at[pl.ds(base + c1 * CH, CH)])
            return carry

        lax.fori_loop(0, NCH // 2, body, 0)

    return gather_kernel(table, idx)


def _seg_scan_body(e_ref, hin_ref, wih_ref, whh_ref, bhn_ref, hout_ref,
                   h_ref):
    g = pl.program_id(0)

    @pl.when(g == 0)
    def _():
        h_ref[...] = hin_ref[...]

    e_blk = e_ref[...].reshape(TB * B, DPAD).astype(jnp.bfloat16)
    gi_all = jnp.dot(e_blk, wih_ref[...],
                     preferred_element_type=jnp.float32).astype(jnp.bfloat16)

    half = jnp.bfloat16(0.5)
    h = h_ref[...]
    for t in range(TB):
        gi = gi_all[t * B:(t + 1) * B]
        gh = jnp.dot(h, whh_ref[...],
                     preferred_element_type=jnp.float32).astype(jnp.bfloat16)
        rt = jnp.tanh(half * (gi[:, :HPAD] + gh[:, :HPAD]))
        zt = jnp.tanh(half * (gi[:, HPAD:2 * HPAD] + gh[:, HPAD:2 * HPAD]))
        hn = gh[:, 2 * HPAD:] + bhn_ref[...]
        n = jnp.tanh(gi[:, 2 * HPAD:] + (half * rt + half) * hn)
        h = (half * ((n + h) + zt * (h - n))).astype(jnp.bfloat16)

    @pl.when(g < NT - 1)
    def _():
        h_ref[...] = h

    @pl.when(g == NT - 1)
    def _():
        hout_ref[...] = h


def _seg_scan(e, h_in, wih, whh, bhn):
    return pl.pallas_call(
        _seg_scan_body,
        grid=(NT,),
        in_specs=[
            pl.BlockSpec((TB, B, DPAD), lambda g: (g, 0, 0)),
            pl.BlockSpec((B, HPAD), lambda g: (0, 0)),
            pl.BlockSpec((DPAD, G3), lambda g: (0, 0)),
            pl.BlockSpec((HPAD, G3), lambda g: (0, 0)),
            pl.BlockSpec((1, HPAD), lambda g: (0, 0)),
        ],
        out_specs=pl.BlockSpec((B, HPAD), lambda g: (0, 0)),
        out_shape=jax.ShapeDtypeStruct((B, HPAD), jnp.bfloat16),
        scratch_shapes=[pltpu.VMEM((B, HPAD), jnp.bfloat16)],
    )(e, h_in, wih, whh, bhn)


def _final_body(h_ref, wd_ref, bd_ref, out_ref):
    a = jnp.tanh(h_ref[...].astype(jnp.float32)).astype(jnp.bfloat16)
    logits = jnp.dot(a, wd_ref[...], preferred_element_type=jnp.float32)
    logits = logits + bd_ref[...]
    m = jnp.max(logits, axis=-1, keepdims=True)
    p = jnp.exp(logits - m)
    p = p / jnp.sum(p, axis=-1, keepdims=True)
    out_ref[...] = p[:, :OUT]


def _final(h, wd, bd):
    return pl.pallas_call(
        _final_body,
        out_shape=jax.ShapeDtypeStruct((B, OUT), jnp.float32),
    )(h, wd, bd)


def _pad_gates_2d(w, rows_to):
    rows = w.shape[0]
    parts = []
    for g in range(3):
        wg = w[:, g * HID:(g + 1) * HID]
        parts.append(jnp.pad(wg, ((0, rows_to - rows), (0, HPAD - HID))))
    return jnp.concatenate(parts, axis=1)


def _pad_gates_1d(b):
    parts = [jnp.pad(b[g * HID:(g + 1) * HID], (0, HPAD - HID))
             for g in range(3)]
    return jnp.concatenate(parts)[None, :]


def kernel(x, emb_table, W_ih, W_hh, b_ih, b_hh, W_dense, b_dense):
    xi = x.astype(jnp.int32)

    table = jnp.concatenate(
        [emb_table,
         jnp.ones((VOCAB, 1), jnp.float32),
         jnp.zeros((VOCAB, DPAD - EMB - 1), jnp.float32)], axis=1)

    b_comb = b_ih + jnp.concatenate(
        [b_hh[:2 * HID], jnp.zeros((HID,), jnp.float32)])
    wih = jnp.concatenate(
        [_pad_gates_2d(W_ih, EMB),
         _pad_gates_1d(b_comb),
         jnp.zeros((DPAD - EMB - 1, G3), jnp.float32)],
        axis=0).astype(jnp.bfloat16)
    whh = _pad_gates_2d(W_hh, HPAD).astype(jnp.bfloat16)
    bhn = jnp.pad(b_hh[2 * HID:],
                  (0, HPAD - HID))[None, :].astype(jnp.bfloat16)
    wd = jnp.pad(W_dense.T,
                 ((0, HPAD - HID), (0, HPAD - OUT))).astype(jnp.bfloat16)
    bd = jnp.pad(b_dense, (0, HPAD - OUT), constant_values=-1e30)[None, :]

    es = []
    for k in range(NSEG):
        idx_k = xi[:, k * LSEG:(k + 1) * LSEG].T.reshape(NW, NCH, CH)
        es.append(_sc_gather(table, idx_k).reshape(LSEG, B, DPAD))

    h = jnp.zeros((B, HPAD), jnp.bfloat16)
    for k in range(NSEG):
        h = _seg_scan(es[k], h, wih, whh, bhn)

    return _final(h, wd, bd)
